# baseline (speedup 1.0000x reference)
import jax
import jax.numpy as jnp
from jax import lax
from jax.experimental import pallas as pl
from jax.experimental.pallas import tpu as pltpu
from jax.experimental.pallas import tpu_sc as plsc

N = 100000
E = 1600000
EPAD = 1638400
NACC = 100224
C = 320
BN = 2000
NB = N // BN
ZSPAN = NACC // 16


def _sc_pass_body(mode, src_hbm, dst_hbm, tab_hbm, out_hbm,
                  sv0, sv1, sv2, sv3, dv0, dv1, dv2, dv3,
                  rv0, rv1, rv2, rv3, acc, g0, g1, g2, g3, i0, i1, i2, i3):
    cid = lax.axis_index("c")
    sid = lax.axis_index("s")
    gather = mode != "deg"

    def _fill(val):
        def fz(i, carry):
            rv0[i] = jnp.full((16,), val, jnp.float32)
            return carry
        lax.fori_loop(0, C, fz, 0)

    _fill(0.0)
    zb = sid * ZSPAN
    for j in range(ZSPAN // C):
        pltpu.sync_copy(rv0, acc.at[pl.ds(zb + j * C, C)])
    rem = ZSPAN % C
    if rem:
        pltpu.sync_copy(rv0.at[pl.ds(0, rem)],
                        acc.at[pl.ds(zb + (ZSPAN // C) * C, rem)])
    if mode == "deg":
        _fill(1.0)
    plsc.subcore_barrier()

    if mode == "prop_f":
        ept = EPAD // 16
        ebase = sid * ept
        sbase = cid * EPAD + ebase
        nch = ept // C
    else:
        w = sid * 2 + cid
        ebase = w * (EPAD // 32)
        sbase = ebase
        nch = (EPAD // 32) // C

    def s_slice(u):
        return src_hbm.at[pl.ds(pl.multiple_of(sbase + u * C, 8), C)]

    def d_slice(u):
        return dst_hbm.at[pl.ds(pl.multiple_of(ebase + u * C, 8), C)]

    sv = (sv0, sv1, sv2, sv3)
    dv = (dv0, dv1, dv2, dv3)
    rv = (rv0, rv1, rv2, rv3)
    gs = (g0, g1, g2, g3)
    isem = (i0, i1, i2, i3)

    for u in range(3):
        if gather:
            pltpu.sync_copy(s_slice(u), sv[u])
        pltpu.sync_copy(d_slice(u), dv[u])
    if gather:
        for u in range(3):
            pltpu.make_async_copy(tab_hbm.at[sv[u]], rv[u], gs[u]).start()
        pltpu.make_async_copy(s_slice(3), sv[3], isem[3]).start()
    pltpu.make_async_copy(d_slice(3), dv[3], isem[3]).start()

    def step(t, b):
        b3 = (b + 3) % 4

        @pl.when(t + 3 < nch)
        def _start_next_gather():
            if gather:
                pltpu.make_async_copy(s_slice(t + 3), sv[b3], isem[b3]).wait()
            pltpu.make_async_copy(d_slice(t + 3), dv[b3], isem[b3]).wait()
            if gather:
                pltpu.make_async_copy(tab_hbm.at[sv[b3]], rv[b3], gs[b3]).start()

        if gather:
            pltpu.make_async_copy(tab_hbm.at[sv[b]], rv[b], gs[b]).wait()
        src_rows = rv[b] if gather else rv0
        pltpu.async_copy(src_rows, acc.at[dv[b]], isem[b], add=True).wait()

        @pl.when(t + 4 < nch)
        def _prefetch_idx():
            if gather:
                pltpu.make_async_copy(s_slice(t + 4), sv[b], isem[b]).start()
            pltpu.make_async_copy(d_slice(t + 4), dv[b], isem[b]).start()

    def quad(p, carry):
        for b in range(4):
            step(4 * p + b, b)
        return carry

    lax.fori_loop(0, nch // 4, quad, 0)
    plsc.subcore_barrier()

    pltpu.sync_copy(acc.at[pl.ds(pl.multiple_of(sid * ZSPAN, 8), ZSPAN)],
                    out_hbm.at[pl.ds(pl.multiple_of(cid * NACC + sid * ZSPAN, 8),
                                     ZSPAN)])


def _make_sc(mode):
    mesh = plsc.VectorSubcoreMesh(core_axis_name="c", subcore_axis_name="s")
    scratch = (
        [pltpu.VMEM((C,), jnp.int32)] * 4 +
        [pltpu.VMEM((C,), jnp.int32)] * 4 +
        [pltpu.VMEM((C, 16), jnp.float32)] * 4 +
        [pltpu.VMEM_SHARED((NACC, 16), jnp.float32)] +
        [pltpu.SemaphoreType.DMA] * 8
    )
    out_t = jax.ShapeDtypeStruct((2 * NACC, 16), jnp.float32)

    if mode == "deg":
        def body(dst_hbm, out_hbm, *bufs):
            _sc_pass_body(mode, None, dst_hbm, None, out_hbm, *bufs)
    else:
        def body(src_hbm, dst_hbm, tab_hbm, out_hbm, *bufs):
            _sc_pass_body(mode, src_hbm, dst_hbm, tab_hbm, out_hbm, *bufs)

    return pl.kernel(body, mesh=mesh, out_type=out_t, scratch_types=scratch,
                     compiler_params=pltpu.CompilerParams(
                         use_tc_tiling_on_sc=False))


_sc_deg = _make_sc("deg")
_sc_prop_e = _make_sc("prop_e")
_sc_prop_f = _make_sc("prop_f")


def _bspec(shape, imap):
    return pl.BlockSpec(shape, imap)


def _row(i):
    return (i, 0)


def _zero(i):
    return (0, 0)


def _row3(i):
    return (i, 0, 0)


def _zero3(i):
    return (0, 0, 0)


def _prep_body(d0, d1, x16, dis_o, xp_o):
    cnt = d0[:, 0] + d1[:, 0] + 1.0
    dis = lax.rsqrt(cnt)
    dis_o[...] = dis[:, None]
    xp_o[...] = x16[...] * dis[:, None]


def _prep(d0, d1, x16):
    return pl.pallas_call(
        _prep_body,
        grid=(NB,),
        in_specs=[_bspec((BN, 16), _row), _bspec((BN, 16), _row),
                  _bspec((BN, 16), _row)],
        out_specs=[_bspec((BN, 1), _row), _bspec((BN, 16), _row)],
        out_shape=[jax.ShapeDtypeStruct((N, 1), jnp.float32),
                   jax.ShapeDtypeStruct((N, 16), jnp.float32)],
    )(d0, d1, x16)


def _conv_out_body(a0, a1, hp, dis, W, b, out_o, s_o, ss_o):
    z = (a0[...] + a1[...] + hp[...]) * dis[...]
    o = jnp.dot(z, W[...], preferred_element_type=jnp.float32) + b[...]
    out_o[...] = o
    s_o[...] = jnp.sum(o, axis=0, keepdims=True)[None]
    ss_o[...] = jnp.sum(o * o, axis=0, keepdims=True)[None]


def _conv_out(a0, a1, hp, dis, W, b, fin, fout):
    return pl.pallas_call(
        _conv_out_body,
        grid=(NB,),
        in_specs=[_bspec((BN, fin), _row), _bspec((BN, fin), _row),
                  _bspec((BN, fin), _row), _bspec((BN, 1), _row),
                  _bspec((fin, fout), _zero), _bspec((1, fout), _zero)],
        out_specs=[_bspec((BN, fout), _row), _bspec((1, 1, fout), _row3),
                   _bspec((1, 1, fout), _row3)],
        out_shape=[jax.ShapeDtypeStruct((N, fout), jnp.float32),
                   jax.ShapeDtypeStruct((NB, 1, fout), jnp.float32),
                   jax.ShapeDtypeStruct((NB, 1, fout), jnp.float32)],
    )(a0, a1, hp, dis, W, b)


def _conv_cat_body(a0, a1, h0, h1, dis, W, b, out_o, s_o, ss_o):
    z = jnp.concatenate([a0[...] + h0[...], a1[...] + h1[...]], axis=1) * dis[...]
    o = jnp.dot(z, W[...], preferred_element_type=jnp.float32) + b[...]
    out_o[...] = o
    s_o[...] = jnp.sum(o, axis=0, keepdims=True)[None]
    ss_o[...] = jnp.sum(o * o, axis=0, keepdims=True)[None]


def _conv_cat(a0, a1, h0, h1, dis, W, b, fout):
    return pl.pallas_call(
        _conv_cat_body,
        grid=(NB,),
        in_specs=[_bspec((BN, 16), _row)] * 4 + [
            _bspec((BN, 1), _row),
            _bspec((32, fout), _zero), _bspec((1, fout), _zero)],
        out_specs=[_bspec((BN, fout), _row), _bspec((1, 1, fout), _row3),
                   _bspec((1, 1, fout), _row3)],
        out_shape=[jax.ShapeDtypeStruct((N, fout), jnp.float32),
                   jax.ShapeDtypeStruct((NB, 1, fout), jnp.float32),
                   jax.ShapeDtypeStruct((NB, 1, fout), jnp.float32)],
    )(a0, a1, h0, h1, dis, W, b)


def _cat_add_body(a0, a1, h0, h1, dis, b, out_o, s_o, ss_o):
    o = jnp.concatenate([a0[...] + h0[...], a1[...] + h1[...]], axis=1) * dis[...] \
        + b[...]
    out_o[...] = o
    s_o[...] = jnp.sum(o, axis=0, keepdims=True)[None]
    ss_o[...] = jnp.sum(o * o, axis=0, keepdims=True)[None]


def _cat_add(a0, a1, h0, h1, dis, b):
    return pl.pallas_call(
        _cat_add_body,
        grid=(NB,),
        in_specs=[_bspec((BN, 16), _row)] * 4 + [
            _bspec((BN, 1), _row), _bspec((1, 32), _zero)],
        out_specs=[_bspec((BN, 32), _row), _bspec((1, 1, 32), _row3),
                   _bspec((1, 1, 32), _row3)],
        out_shape=[jax.ShapeDtypeStruct((N, 32), jnp.float32),
                   jax.ShapeDtypeStruct((NB, 1, 32), jnp.float32),
                   jax.ShapeDtypeStruct((NB, 1, 32), jnp.float32)],
    )(a0, a1, h0, h1, dis, b)


def _bn_relu(o_ref, s, ss, g, be):
    m = jnp.sum(s[...], axis=(0, 1)) / N
    v = jnp.sum(ss[...], axis=(0, 1)) / N - m * m
    return jnp.maximum((o_ref[...] - m) * lax.rsqrt(v + 1e-5) * g[...] + be[...],
                       0.0)


def _bn_split_nw_body(out_k, s, ss, g, be, dis, h0_o, h1_o, pmax_o, psum_o):
    xk = _bn_relu(out_k, s, ss, g, be)
    pmax_o[...] = jnp.max(xk, axis=0, keepdims=True)[None]
    psum_o[...] = jnp.sum(xk, axis=0, keepdims=True)[None]
    h = xk * dis[...]
    h0_o[...] = h[:, :16]
    h1_o[...] = h[:, 16:]


def _bn_split_w_body(out_k, s, ss, g, be, dis, W, h0_o, h1_o, pmax_o, psum_o):
    xk = _bn_relu(out_k, s, ss, g, be)
    pmax_o[...] = jnp.max(xk, axis=0, keepdims=True)[None]
    psum_o[...] = jnp.sum(xk, axis=0, keepdims=True)[None]
    h = jnp.dot(xk, W[...], preferred_element_type=jnp.float32) * dis[...]
    h0_o[...] = h[:, :16]
    h1_o[...] = h[:, 16:]


def _bn_split(out_k, s, ss, g, be, dis, W, f):
    if W is None:
        body = _bn_split_nw_body
        wspecs = []
        args = (out_k, s, ss, g, be, dis)
    else:
        body = _bn_split_w_body
        wspecs = [_bspec((f, 32), _zero)]
        args = (out_k, s, ss, g, be, dis, W)
    return pl.pallas_call(
        body,
        grid=(NB,),
        in_specs=[_bspec((BN, f), _row), _bspec((NB, 1, f), _zero3),
                  _bspec((NB, 1, f), _zero3), _bspec((1, f), _zero),
                  _bspec((1, f), _zero), _bspec((BN, 1), _row)] + wspecs,
        out_specs=[_bspec((BN, 16), _row), _bspec((BN, 16), _row),
                   _bspec((1, 1, f), _row3), _bspec((1, 1, f), _row3)],
        out_shape=[jax.ShapeDtypeStruct((N, 16), jnp.float32),
                   jax.ShapeDtypeStruct((N, 16), jnp.float32),
                   jax.ShapeDtypeStruct((NB, 1, f), jnp.float32),
                   jax.ShapeDtypeStruct((NB, 1, f), jnp.float32)],
    )(*args)


def _bn_pool_body(out_k, s, ss, g, be, pmax_o, psum_o):
    xk = _bn_relu(out_k, s, ss, g, be)
    pmax_o[...] = jnp.max(xk, axis=0, keepdims=True)[None]
    psum_o[...] = jnp.sum(xk, axis=0, keepdims=True)[None]


def _bn_pool(out_k, s, ss, g, be, f):
    return pl.pallas_call(
        _bn_pool_body,
        grid=(NB,),
        in_specs=[_bspec((BN, f), _row), _bspec((NB, 1, f), _zero3),
                  _bspec((NB, 1, f), _zero3), _bspec((1, f), _zero),
                  _bspec((1, f), _zero)],
        out_specs=[_bspec((1, 1, f), _row3), _bspec((1, 1, f), _row3)],
        out_shape=[jax.ShapeDtypeStruct((NB, 1, f), jnp.float32),
                   jax.ShapeDtypeStruct((NB, 1, f), jnp.float32)],
    )(out_k, s, ss, g, be)


def _head_body(pm1, ps1, pm2, ps2, pm3, ps3, Wc1, bc1, lng, lnb, Wc2, bc2,
               out_o):
    mx = jnp.concatenate([jnp.max(pm1[...], axis=(0, 1)),
                          jnp.max(pm2[...], axis=(0, 1)),
                          jnp.max(pm3[...], axis=(0, 1))])
    mn = jnp.concatenate([jnp.sum(ps1[...], axis=(0, 1)),
                          jnp.sum(ps2[...], axis=(0, 1)),
                          jnp.sum(ps3[...], axis=(0, 1))]) / N
    pooled = jnp.concatenate([mx, mn])[None, :]
    h = jnp.dot(pooled, Wc1[...], preferred_element_type=jnp.float32) + bc1[...]
    m = jnp.mean(h, axis=-1, keepdims=True)
    v = jnp.mean((h - m) ** 2, axis=-1, keepdims=True)
    h = jnp.maximum((h - m) * lax.rsqrt(v + 1e-5) * lng[...] + lnb[...], 0.0)
    out_o[...] = jnp.dot(h, Wc2[...], preferred_element_type=jnp.float32) \
        + bc2[...]


def _head(pm1, ps1, pm2, ps2, pm3, ps3, Wc1, bc1, lng, lnb, Wc2, bc2):
    return pl.pallas_call(
        _head_body,
        out_shape=jax.ShapeDtypeStruct((1, 6), jnp.float32),
    )(pm1, ps1, pm2, ps2, pm3, ps3, Wc1, bc1, lng, lnb, Wc2, bc2)


def kernel(adj, x, W1, b1, W2, b2, W3, b3, g1, be1, g2, be2, g3, be3,
           Wc1, bc1, lng, lnb, Wc2, bc2):
    src, dst = adj[0], adj[1]
    npad = EPAD - E
    srcp = jnp.concatenate([src, jnp.zeros((npad,), jnp.int32)])
    dstp = jnp.concatenate([dst, jnp.full((npad,), N, jnp.int32)])
    src2 = jnp.concatenate([srcp, srcp + N])
    dst2d = dstp

    r = lambda a: a[None, :]

    halves = lambda a: (a[:N], a[NACC:NACC + N])

    degf = _sc_deg(dst2d)
    x16 = jnp.pad(x, ((0, 0), (0, 13)))
    dis, xp16 = _prep(*halves(degf), x16)

    agg1 = _sc_prop_e(srcp, dst2d, xp16)
    W1p = jnp.pad(W1, ((0, 13), (0, 0)))
    out1, s1, ss1 = _conv_out(*halves(agg1), xp16, dis, W1p, r(b1), 16, 32)
    h2a, h2b, pm1, ps1 = _bn_split(out1, s1, ss1, r(g1), r(be1), dis, None, 32)

    h2 = jnp.concatenate([h2a, h2b], axis=0)
    agg2 = _sc_prop_f(src2, dst2d, h2)
    out2, s2, ss2 = _conv_cat(*halves(agg2), h2a, h2b, dis, W2, r(b2), 64)
    h3a, h3b, pm2, ps2 = _bn_split(out2, s2, ss2, r(g2), r(be2), dis, W3, 64)

    h3 = jnp.concatenate([h3a, h3b], axis=0)
    agg3 = _sc_prop_f(src2, dst2d, h3)
    out3, s3, ss3 = _cat_add(*halves(agg3), h3a, h3b, dis, r(b3))
    pm3, ps3 = _bn_pool(out3, s3, ss3, r(g3), r(be3), 32)

    return _head(pm1, ps1, pm2, ps2, pm3, ps3,
                 Wc1, r(bc1), r(lng), r(lnb), Wc2, r(bc2))


# speedup vs baseline: 1.0420x; 1.0420x over previous
import jax
import jax.numpy as jnp
from jax import lax
from jax.experimental import pallas as pl
from jax.experimental.pallas import tpu as pltpu
from jax.experimental.pallas import tpu_sc as plsc

N = 100000
E = 1600000
EPAD = 1638400
NACC = 100224
C = 800
D = 2
BN = 2000
NB = N // BN
ZSPAN = NACC // 16


def _sc_pass_body(mode, src_hbm, dst_hbm, tab_hbm, out_hbm, *bufs):
    sv = bufs[0:D]
    dv = bufs[D:2 * D]
    rv = bufs[2 * D:3 * D]
    acc = bufs[3 * D]
    gs = bufs[3 * D + 1:3 * D + 1 + D]
    isem = bufs[3 * D + 1 + D:3 * D + 1 + 2 * D]
    rv0 = rv[0]
    cid = lax.axis_index("c")
    sid = lax.axis_index("s")
    gather = mode != "deg"

    def _fill(val):
        def fz(i, carry):
            rv0[i] = jnp.full((16,), val, jnp.float32)
            return carry
        lax.fori_loop(0, C, fz, 0)

    _fill(0.0)
    zb = sid * ZSPAN
    for j in range(ZSPAN // C):
        pltpu.sync_copy(rv0, acc.at[pl.ds(zb + j * C, C)])
    rem = ZSPAN % C
    if rem:
        pltpu.sync_copy(rv0.at[pl.ds(0, rem)],
                        acc.at[pl.ds(zb + (ZSPAN // C) * C, rem)])
    if mode == "deg":
        _fill(1.0)
    plsc.subcore_barrier()

    if mode == "prop_f":
        ept = EPAD // 16
        ebase = sid * ept
        sbase = cid * EPAD + ebase
        nch = ept // C
    else:
        w = sid * 2 + cid
        ebase = w * (EPAD // 32)
        sbase = ebase
        nch = (EPAD // 32) // C

    def s_slice(u):
        return src_hbm.at[pl.ds(pl.multiple_of(sbase + u * C, 8), C)]

    def d_slice(u):
        return dst_hbm.at[pl.ds(pl.multiple_of(ebase + u * C, 8), C)]

    for u in range(D - 1):
        if gather:
            pltpu.sync_copy(s_slice(u), sv[u])
        pltpu.sync_copy(d_slice(u), dv[u])
    if gather:
        for u in range(D - 1):
            pltpu.make_async_copy(tab_hbm.at[sv[u]], rv[u], gs[u]).start()
        pltpu.make_async_copy(s_slice(D - 1), sv[D - 1], isem[D - 1]).start()
    pltpu.make_async_copy(d_slice(D - 1), dv[D - 1], isem[D - 1]).start()

    def step(t, b):
        bn = (b + D - 1) % D

        @pl.when(t + D - 1 < nch)
        def _start_next_gather():
            if gather:
                pltpu.make_async_copy(s_slice(t + D - 1), sv[bn], isem[bn]).wait()
            pltpu.make_async_copy(d_slice(t + D - 1), dv[bn], isem[bn]).wait()
            if gather:
                pltpu.make_async_copy(tab_hbm.at[sv[bn]], rv[bn], gs[bn]).start()

        if gather:
            pltpu.make_async_copy(tab_hbm.at[sv[b]], rv[b], gs[b]).wait()
        src_rows = rv[b] if gather else rv0
        pltpu.async_copy(src_rows, acc.at[dv[b]], isem[b], add=True).wait()

        @pl.when(t + D < nch)
        def _prefetch_idx():
            if gather:
                pltpu.make_async_copy(s_slice(t + D), sv[b], isem[b]).start()
            pltpu.make_async_copy(d_slice(t + D), dv[b], isem[b]).start()

    def rotation(p, carry):
        for b in range(D):
            step(D * p + b, b)
        return carry

    lax.fori_loop(0, nch // D, rotation, 0)
    plsc.subcore_barrier()

    pltpu.sync_copy(acc.at[pl.ds(pl.multiple_of(sid * ZSPAN, 8), ZSPAN)],
                    out_hbm.at[pl.ds(pl.multiple_of(cid * NACC + sid * ZSPAN, 8),
                                     ZSPAN)])


def _make_sc(mode):
    mesh = plsc.VectorSubcoreMesh(core_axis_name="c", subcore_axis_name="s")
    scratch = (
        [pltpu.VMEM((C,), jnp.int32)] * D +
        [pltpu.VMEM((C,), jnp.int32)] * D +
        [pltpu.VMEM((C, 16), jnp.float32)] * D +
        [pltpu.VMEM_SHARED((NACC, 16), jnp.float32)] +
        [pltpu.SemaphoreType.DMA] * (2 * D)
    )
    out_t = jax.ShapeDtypeStruct((2 * NACC, 16), jnp.float32)

    if mode == "deg":
        def body(dst_hbm, out_hbm, *bufs):
            _sc_pass_body(mode, None, dst_hbm, None, out_hbm, *bufs)
    else:
        def body(src_hbm, dst_hbm, tab_hbm, out_hbm, *bufs):
            _sc_pass_body(mode, src_hbm, dst_hbm, tab_hbm, out_hbm, *bufs)

    return pl.kernel(body, mesh=mesh, out_type=out_t, scratch_types=scratch,
                     compiler_params=pltpu.CompilerParams(
                         use_tc_tiling_on_sc=False))


_sc_deg = _make_sc("deg")
_sc_prop_e = _make_sc("prop_e")
_sc_prop_f = _make_sc("prop_f")


def _bspec(shape, imap):
    return pl.BlockSpec(shape, imap)


def _row(i):
    return (i, 0)


def _zero(i):
    return (0, 0)


def _row3(i):
    return (i, 0, 0)


def _zero3(i):
    return (0, 0, 0)


def _prep_body(d0, d1, x16, dis_o, xp_o):
    cnt = d0[:, 0] + d1[:, 0] + 1.0
    dis = lax.rsqrt(cnt)
    dis_o[...] = dis[:, None]
    xp_o[...] = x16[...] * dis[:, None]


def _prep(d0, d1, x16):
    return pl.pallas_call(
        _prep_body,
        grid=(NB,),
        in_specs=[_bspec((BN, 16), _row), _bspec((BN, 16), _row),
                  _bspec((BN, 16), _row)],
        out_specs=[_bspec((BN, 1), _row), _bspec((BN, 16), _row)],
        out_shape=[jax.ShapeDtypeStruct((N, 1), jnp.float32),
                   jax.ShapeDtypeStruct((N, 16), jnp.float32)],
    )(d0, d1, x16)


def _conv_out_body(a0, a1, hp, dis, W, b, out_o, s_o, ss_o):
    z = (a0[...] + a1[...] + hp[...]) * dis[...]
    o = jnp.dot(z, W[...], preferred_element_type=jnp.float32) + b[...]
    out_o[...] = o
    s_o[...] = jnp.sum(o, axis=0, keepdims=True)[None]
    ss_o[...] = jnp.sum(o * o, axis=0, keepdims=True)[None]


def _conv_out(a0, a1, hp, dis, W, b, fin, fout):
    return pl.pallas_call(
        _conv_out_body,
        grid=(NB,),
        in_specs=[_bspec((BN, fin), _row), _bspec((BN, fin), _row),
                  _bspec((BN, fin), _row), _bspec((BN, 1), _row),
                  _bspec((fin, fout), _zero), _bspec((1, fout), _zero)],
        out_specs=[_bspec((BN, fout), _row), _bspec((1, 1, fout), _row3),
                   _bspec((1, 1, fout), _row3)],
        out_shape=[jax.ShapeDtypeStruct((N, fout), jnp.float32),
                   jax.ShapeDtypeStruct((NB, 1, fout), jnp.float32),
                   jax.ShapeDtypeStruct((NB, 1, fout), jnp.float32)],
    )(a0, a1, hp, dis, W, b)


def _conv_cat_body(a0, a1, h0, h1, dis, W, b, out_o, s_o, ss_o):
    z = jnp.concatenate([a0[...] + h0[...], a1[...] + h1[...]], axis=1) * dis[...]
    o = jnp.dot(z, W[...], preferred_element_type=jnp.float32) + b[...]
    out_o[...] = o
    s_o[...] = jnp.sum(o, axis=0, keepdims=True)[None]
    ss_o[...] = jnp.sum(o * o, axis=0, keepdims=True)[None]


def _conv_cat(a0, a1, h0, h1, dis, W, b, fout):
    return pl.pallas_call(
        _conv_cat_body,
        grid=(NB,),
        in_specs=[_bspec((BN, 16), _row)] * 4 + [
            _bspec((BN, 1), _row),
            _bspec((32, fout), _zero), _bspec((1, fout), _zero)],
        out_specs=[_bspec((BN, fout), _row), _bspec((1, 1, fout), _row3),
                   _bspec((1, 1, fout), _row3)],
        out_shape=[jax.ShapeDtypeStruct((N, fout), jnp.float32),
                   jax.ShapeDtypeStruct((NB, 1, fout), jnp.float32),
                   jax.ShapeDtypeStruct((NB, 1, fout), jnp.float32)],
    )(a0, a1, h0, h1, dis, W, b)


def _cat_add_body(a0, a1, h0, h1, dis, b, out_o, s_o, ss_o):
    o = jnp.concatenate([a0[...] + h0[...], a1[...] + h1[...]], axis=1) * dis[...] \
        + b[...]
    out_o[...] = o
    s_o[...] = jnp.sum(o, axis=0, keepdims=True)[None]
    ss_o[...] = jnp.sum(o * o, axis=0, keepdims=True)[None]


def _cat_add(a0, a1, h0, h1, dis, b):
    return pl.pallas_call(
        _cat_add_body,
        grid=(NB,),
        in_specs=[_bspec((BN, 16), _row)] * 4 + [
            _bspec((BN, 1), _row), _bspec((1, 32), _zero)],
        out_specs=[_bspec((BN, 32), _row), _bspec((1, 1, 32), _row3),
                   _bspec((1, 1, 32), _row3)],
        out_shape=[jax.ShapeDtypeStruct((N, 32), jnp.float32),
                   jax.ShapeDtypeStruct((NB, 1, 32), jnp.float32),
                   jax.ShapeDtypeStruct((NB, 1, 32), jnp.float32)],
    )(a0, a1, h0, h1, dis, b)


def _bn_relu(o_ref, s, ss, g, be):
    m = jnp.sum(s[...], axis=(0, 1)) / N
    v = jnp.sum(ss[...], axis=(0, 1)) / N - m * m
    return jnp.maximum((o_ref[...] - m) * lax.rsqrt(v + 1e-5) * g[...] + be[...],
                       0.0)


def _bn_split_nw_body(out_k, s, ss, g, be, dis, h0_o, h1_o, pmax_o, psum_o):
    xk = _bn_relu(out_k, s, ss, g, be)
    pmax_o[...] = jnp.max(xk, axis=0, keepdims=True)[None]
    psum_o[...] = jnp.sum(xk, axis=0, keepdims=True)[None]
    h = xk * dis[...]
    h0_o[...] = h[:, :16]
    h1_o[...] = h[:, 16:]


def _bn_split_w_body(out_k, s, ss, g, be, dis, W, h0_o, h1_o, pmax_o, psum_o):
    xk = _bn_relu(out_k, s, ss, g, be)
    pmax_o[...] = jnp.max(xk, axis=0, keepdims=True)[None]
    psum_o[...] = jnp.sum(xk, axis=0, keepdims=True)[None]
    h = jnp.dot(xk, W[...], preferred_element_type=jnp.float32) * dis[...]
    h0_o[...] = h[:, :16]
    h1_o[...] = h[:, 16:]


def _bn_split(out_k, s, ss, g, be, dis, W, f):
    if W is None:
        body = _bn_split_nw_body
        wspecs = []
        args = (out_k, s, ss, g, be, dis)
    else:
        body = _bn_split_w_body
        wspecs = [_bspec((f, 32), _zero)]
        args = (out_k, s, ss, g, be, dis, W)
    return pl.pallas_call(
        body,
        grid=(NB,),
        in_specs=[_bspec((BN, f), _row), _bspec((NB, 1, f), _zero3),
                  _bspec((NB, 1, f), _zero3), _bspec((1, f), _zero),
                  _bspec((1, f), _zero), _bspec((BN, 1), _row)] + wspecs,
        out_specs=[_bspec((BN, 16), _row), _bspec((BN, 16), _row),
                   _bspec((1, 1, f), _row3), _bspec((1, 1, f), _row3)],
        out_shape=[jax.ShapeDtypeStruct((N, 16), jnp.float32),
                   jax.ShapeDtypeStruct((N, 16), jnp.float32),
                   jax.ShapeDtypeStruct((NB, 1, f), jnp.float32),
                   jax.ShapeDtypeStruct((NB, 1, f), jnp.float32)],
    )(*args)


def _bn_pool_body(out_k, s, ss, g, be, pmax_o, psum_o):
    xk = _bn_relu(out_k, s, ss, g, be)
    pmax_o[...] = jnp.max(xk, axis=0, keepdims=True)[None]
    psum_o[...] = jnp.sum(xk, axis=0, keepdims=True)[None]


def _bn_pool(out_k, s, ss, g, be, f):
    return pl.pallas_call(
        _bn_pool_body,
        grid=(NB,),
        in_specs=[_bspec((BN, f), _row), _bspec((NB, 1, f), _zero3),
                  _bspec((NB, 1, f), _zero3), _bspec((1, f), _zero),
                  _bspec((1, f), _zero)],
        out_specs=[_bspec((1, 1, f), _row3), _bspec((1, 1, f), _row3)],
        out_shape=[jax.ShapeDtypeStruct((NB, 1, f), jnp.float32),
                   jax.ShapeDtypeStruct((NB, 1, f), jnp.float32)],
    )(out_k, s, ss, g, be)


def _head_body(pm1, ps1, pm2, ps2, pm3, ps3, Wc1, bc1, lng, lnb, Wc2, bc2,
               out_o):
    mx = jnp.concatenate([jnp.max(pm1[...], axis=(0, 1)),
                          jnp.max(pm2[...], axis=(0, 1)),
                          jnp.max(pm3[...], axis=(0, 1))])
    mn = jnp.concatenate([jnp.sum(ps1[...], axis=(0, 1)),
                          jnp.sum(ps2[...], axis=(0, 1)),
                          jnp.sum(ps3[...], axis=(0, 1))]) / N
    pooled = jnp.concatenate([mx, mn])[None, :]
    h = jnp.dot(pooled, Wc1[...], preferred_element_type=jnp.float32) + bc1[...]
    m = jnp.mean(h, axis=-1, keepdims=True)
    v = jnp.mean((h - m) ** 2, axis=-1, keepdims=True)
    h = jnp.maximum((h - m) * lax.rsqrt(v + 1e-5) * lng[...] + lnb[...], 0.0)
    out_o[...] = jnp.dot(h, Wc2[...], preferred_element_type=jnp.float32) \
        + bc2[...]


def _head(pm1, ps1, pm2, ps2, pm3, ps3, Wc1, bc1, lng, lnb, Wc2, bc2):
    return pl.pallas_call(
        _head_body,
        out_shape=jax.ShapeDtypeStruct((1, 6), jnp.float32),
    )(pm1, ps1, pm2, ps2, pm3, ps3, Wc1, bc1, lng, lnb, Wc2, bc2)


def kernel(adj, x, W1, b1, W2, b2, W3, b3, g1, be1, g2, be2, g3, be3,
           Wc1, bc1, lng, lnb, Wc2, bc2):
    src, dst = adj[0], adj[1]
    npad = EPAD - E
    srcp = jnp.concatenate([src, jnp.zeros((npad,), jnp.int32)])
    dstp = jnp.concatenate([dst, jnp.full((npad,), N, jnp.int32)])
    src2 = jnp.concatenate([srcp, srcp + N])
    dst2d = dstp

    r = lambda a: a[None, :]

    halves = lambda a: (a[:N], a[NACC:NACC + N])

    degf = _sc_deg(dst2d)
    x16 = jnp.pad(x, ((0, 0), (0, 13)))
    dis, xp16 = _prep(*halves(degf), x16)

    agg1 = _sc_prop_e(srcp, dst2d, xp16)
    W1p = jnp.pad(W1, ((0, 13), (0, 0)))
    out1, s1, ss1 = _conv_out(*halves(agg1), xp16, dis, W1p, r(b1), 16, 32)
    h2a, h2b, pm1, ps1 = _bn_split(out1, s1, ss1, r(g1), r(be1), dis, None, 32)

    h2 = jnp.concatenate([h2a, h2b], axis=0)
    agg2 = _sc_prop_f(src2, dst2d, h2)
    out2, s2, ss2 = _conv_cat(*halves(agg2), h2a, h2b, dis, W2, r(b2), 64)
    h3a, h3b, pm2, ps2 = _bn_split(out2, s2, ss2, r(g2), r(be2), dis, W3, 64)

    h3 = jnp.concatenate([h3a, h3b], axis=0)
    agg3 = _sc_prop_f(src2, dst2d, h3)
    out3, s3, ss3 = _cat_add(*halves(agg3), h3a, h3b, dis, r(b3))
    pm3, ps3 = _bn_pool(out3, s3, ss3, r(g3), r(be3), 32)

    return _head(pm1, ps1, pm2, ps2, pm3, ps3,
                 Wc1, r(bc1), r(lng), r(lnb), Wc2, r(bc2))


# speedup vs baseline: 1.1758x; 1.1285x over previous
import jax
import jax.numpy as jnp
from jax import lax
from jax.experimental import pallas as pl
from jax.experimental.pallas import tpu as pltpu
from jax.experimental.pallas import tpu_sc as plsc

N = 100000
E = 1600000
EPAD = 1638400
NACC = 100224
C = 800
D = 2
BN = 2000
NB = N // BN
ZSPAN = NACC // 16


def _sc_pass_body(mode, src_hbm, dst_hbm, tab_hbm, out_lo, out_hi, *bufs):
    sv = bufs[0:D]
    dv = bufs[D:2 * D]
    rv = bufs[2 * D:3 * D]
    acc = bufs[3 * D]
    gs = bufs[3 * D + 1:3 * D + 1 + D]
    isem = bufs[3 * D + 1 + D:3 * D + 1 + 2 * D]
    rv0 = rv[0]
    cid = lax.axis_index("c")
    sid = lax.axis_index("s")
    gather = mode != "deg"

    def _fill(val):
        def fz(i, carry):
            rv0[i] = jnp.full((16,), val, jnp.float32)
            return carry
        lax.fori_loop(0, C, fz, 0)

    _fill(0.0)
    zb = sid * ZSPAN
    for j in range(ZSPAN // C):
        pltpu.sync_copy(rv0, acc.at[pl.ds(zb + j * C, C)])
    rem = ZSPAN % C
    if rem:
        pltpu.sync_copy(rv0.at[pl.ds(0, rem)],
                        acc.at[pl.ds(zb + (ZSPAN // C) * C, rem)])
    if mode == "deg":
        _fill(1.0)
    plsc.subcore_barrier()

    if mode == "prop_f":
        ept = EPAD // 16
        ebase = sid * ept
        sbase = cid * EPAD + ebase
        nch = ept // C
    else:
        w = sid * 2 + cid
        ebase = w * (EPAD // 32)
        sbase = ebase
        nch = (EPAD // 32) // C

    def s_slice(u):
        return src_hbm.at[pl.ds(pl.multiple_of(sbase + u * C, 8), C)]

    def d_slice(u):
        return dst_hbm.at[pl.ds(pl.multiple_of(ebase + u * C, 8), C)]

    for u in range(D - 1):
        if gather:
            pltpu.sync_copy(s_slice(u), sv[u])
        pltpu.sync_copy(d_slice(u), dv[u])
    if gather:
        for u in range(D - 1):
            pltpu.make_async_copy(tab_hbm.at[sv[u]], rv[u], gs[u]).start()
        pltpu.make_async_copy(s_slice(D - 1), sv[D - 1], isem[D - 1]).start()
    pltpu.make_async_copy(d_slice(D - 1), dv[D - 1], isem[D - 1]).start()

    def step(t, b):
        bn = (b + D - 1) % D

        @pl.when(t + D - 1 < nch)
        def _start_next_gather():
            if gather:
                pltpu.make_async_copy(s_slice(t + D - 1), sv[bn], isem[bn]).wait()
            pltpu.make_async_copy(d_slice(t + D - 1), dv[bn], isem[bn]).wait()
            if gather:
                pltpu.make_async_copy(tab_hbm.at[sv[bn]], rv[bn], gs[bn]).start()

        if gather:
            pltpu.make_async_copy(tab_hbm.at[sv[b]], rv[b], gs[b]).wait()
        src_rows = rv[b] if gather else rv0
        pltpu.async_copy(src_rows, acc.at[dv[b]], isem[b], add=True).wait()

        @pl.when(t + D < nch)
        def _prefetch_idx():
            if gather:
                pltpu.make_async_copy(s_slice(t + D), sv[b], isem[b]).start()
            pltpu.make_async_copy(d_slice(t + D), dv[b], isem[b]).start()

    def rotation(p, carry):
        for b in range(D):
            step(D * p + b, b)
        return carry

    lax.fori_loop(0, nch // D, rotation, 0)
    plsc.subcore_barrier()

    zoff = pl.multiple_of(sid * ZSPAN, 8)

    @pl.when(cid == 0)
    def _wb_lo():
        pltpu.sync_copy(acc.at[pl.ds(zoff, ZSPAN)],
                        out_lo.at[pl.ds(zoff, ZSPAN)])

    @pl.when(cid == 1)
    def _wb_hi():
        pltpu.sync_copy(acc.at[pl.ds(zoff, ZSPAN)],
                        out_hi.at[pl.ds(zoff, ZSPAN)])


def _make_sc(mode):
    mesh = plsc.VectorSubcoreMesh(core_axis_name="c", subcore_axis_name="s")
    scratch = (
        [pltpu.VMEM((C,), jnp.int32)] * D +
        [pltpu.VMEM((C,), jnp.int32)] * D +
        [pltpu.VMEM((C, 16), jnp.float32)] * D +
        [pltpu.VMEM_SHARED((NACC, 16), jnp.float32)] +
        [pltpu.SemaphoreType.DMA] * (2 * D)
    )
    out_t = [jax.ShapeDtypeStruct((NACC, 16), jnp.float32)] * 2

    if mode == "deg":
        def body(dst_hbm, out_lo, out_hi, *bufs):
            _sc_pass_body(mode, None, dst_hbm, None, out_lo, out_hi, *bufs)
    else:
        def body(src_hbm, dst_hbm, tab_hbm, out_lo, out_hi, *bufs):
            _sc_pass_body(mode, src_hbm, dst_hbm, tab_hbm, out_lo, out_hi,
                          *bufs)

    return pl.kernel(body, mesh=mesh, out_type=out_t, scratch_types=scratch,
                     compiler_params=pltpu.CompilerParams(
                         use_tc_tiling_on_sc=False))


_sc_deg = _make_sc("deg")
_sc_prop_e = _make_sc("prop_e")
_sc_prop_f = _make_sc("prop_f")


def _bspec(shape, imap):
    return pl.BlockSpec(shape, imap)


def _row(i):
    return (i, 0)


def _zero(i):
    return (0, 0)


def _row3(i):
    return (i, 0, 0)


def _zero3(i):
    return (0, 0, 0)


def _prep_body(d0, d1, x, dis_o, xp_o):
    cnt = d0[:, 0] + d1[:, 0] + 1.0
    dis = lax.rsqrt(cnt)
    dis_o[...] = dis[:, None]
    xp_o[...] = jnp.concatenate(
        [x[...] * dis[:, None], jnp.zeros((BN, 13), jnp.float32)], axis=1)


def _prep(d0, d1, x):
    return pl.pallas_call(
        _prep_body,
        grid=(NB,),
        in_specs=[_bspec((BN, 16), _row), _bspec((BN, 16), _row),
                  _bspec((BN, 3), _row)],
        out_specs=[_bspec((BN, 1), _row), _bspec((BN, 16), _row)],
        out_shape=[jax.ShapeDtypeStruct((N, 1), jnp.float32),
                   jax.ShapeDtypeStruct((N, 16), jnp.float32)],
    )(d0, d1, x)


def _conv_out_body(a0, a1, hp, dis, W, b, out_o, s_o, ss_o):
    z = (a0[...] + a1[...] + hp[...]) * dis[...]
    o = jnp.dot(z, W[...], preferred_element_type=jnp.float32) + b[...]
    out_o[...] = o
    s_o[...] = jnp.sum(o, axis=0, keepdims=True)[None]
    ss_o[...] = jnp.sum(o * o, axis=0, keepdims=True)[None]


def _conv_out(a0, a1, hp, dis, W, b, fin, fout):
    return pl.pallas_call(
        _conv_out_body,
        grid=(NB,),
        in_specs=[_bspec((BN, fin), _row), _bspec((BN, fin), _row),
                  _bspec((BN, fin), _row), _bspec((BN, 1), _row),
                  _bspec((fin, fout), _zero), _bspec((1, fout), _zero)],
        out_specs=[_bspec((BN, fout), _row), _bspec((1, 1, fout), _row3),
                   _bspec((1, 1, fout), _row3)],
        out_shape=[jax.ShapeDtypeStruct((N, fout), jnp.float32),
                   jax.ShapeDtypeStruct((NB, 1, fout), jnp.float32),
                   jax.ShapeDtypeStruct((NB, 1, fout), jnp.float32)],
    )(a0, a1, hp, dis, W, b)


def _conv_cat_body(a0, a1, h, dis, W, b, out_o, s_o, ss_o):
    z = (jnp.concatenate([a0[...], a1[...]], axis=1) + h[...]) * dis[...]
    o = jnp.dot(z, W[...], preferred_element_type=jnp.float32) + b[...]
    out_o[...] = o
    s_o[...] = jnp.sum(o, axis=0, keepdims=True)[None]
    ss_o[...] = jnp.sum(o * o, axis=0, keepdims=True)[None]


def _conv_cat(a0, a1, h, dis, W, b, fout):
    return pl.pallas_call(
        _conv_cat_body,
        grid=(NB,),
        in_specs=[_bspec((BN, 16), _row)] * 2 + [
            _bspec((BN, 32), _row), _bspec((BN, 1), _row),
            _bspec((32, fout), _zero), _bspec((1, fout), _zero)],
        out_specs=[_bspec((BN, fout), _row), _bspec((1, 1, fout), _row3),
                   _bspec((1, 1, fout), _row3)],
        out_shape=[jax.ShapeDtypeStruct((N, fout), jnp.float32),
                   jax.ShapeDtypeStruct((NB, 1, fout), jnp.float32),
                   jax.ShapeDtypeStruct((NB, 1, fout), jnp.float32)],
    )(a0, a1, h, dis, W, b)


def _cat_add_body(a0, a1, h, dis, b, out_o, s_o, ss_o):
    o = (jnp.concatenate([a0[...], a1[...]], axis=1) + h[...]) * dis[...] \
        + b[...]
    out_o[...] = o
    s_o[...] = jnp.sum(o, axis=0, keepdims=True)[None]
    ss_o[...] = jnp.sum(o * o, axis=0, keepdims=True)[None]


def _cat_add(a0, a1, h, dis, b):
    return pl.pallas_call(
        _cat_add_body,
        grid=(NB,),
        in_specs=[_bspec((BN, 16), _row)] * 2 + [
            _bspec((BN, 32), _row), _bspec((BN, 1), _row),
            _bspec((1, 32), _zero)],
        out_specs=[_bspec((BN, 32), _row), _bspec((1, 1, 32), _row3),
                   _bspec((1, 1, 32), _row3)],
        out_shape=[jax.ShapeDtypeStruct((N, 32), jnp.float32),
                   jax.ShapeDtypeStruct((NB, 1, 32), jnp.float32),
                   jax.ShapeDtypeStruct((NB, 1, 32), jnp.float32)],
    )(a0, a1, h, dis, b)


def _bn_relu(o_ref, s, ss, g, be):
    m = jnp.sum(s[...], axis=(0, 1)) / N
    v = jnp.sum(ss[...], axis=(0, 1)) / N - m * m
    return jnp.maximum((o_ref[...] - m) * lax.rsqrt(v + 1e-5) * g[...] + be[...],
                       0.0)


def _bn_split_nw_body(out_k, s, ss, g, be, dis, h_o, pmax_o, psum_o):
    xk = _bn_relu(out_k, s, ss, g, be)
    pmax_o[...] = jnp.max(xk, axis=0, keepdims=True)[None]
    psum_o[...] = jnp.sum(xk, axis=0, keepdims=True)[None]
    h_o[...] = xk * dis[...]


def _bn_split_w_body(out_k, s, ss, g, be, dis, W, h_o, pmax_o, psum_o):
    xk = _bn_relu(out_k, s, ss, g, be)
    pmax_o[...] = jnp.max(xk, axis=0, keepdims=True)[None]
    psum_o[...] = jnp.sum(xk, axis=0, keepdims=True)[None]
    h_o[...] = jnp.dot(xk, W[...], preferred_element_type=jnp.float32) * dis[...]


def _bn_split(out_k, s, ss, g, be, dis, W, f):
    if W is None:
        body = _bn_split_nw_body
        wspecs = []
        args = (out_k, s, ss, g, be, dis)
    else:
        body = _bn_split_w_body
        wspecs = [_bspec((f, 32), _zero)]
        args = (out_k, s, ss, g, be, dis, W)
    return pl.pallas_call(
        body,
        grid=(NB,),
        in_specs=[_bspec((BN, f), _row), _bspec((NB, 1, f), _zero3),
                  _bspec((NB, 1, f), _zero3), _bspec((1, f), _zero),
                  _bspec((1, f), _zero), _bspec((BN, 1), _row)] + wspecs,
        out_specs=[_bspec((BN, 32), _row),
                   _bspec((1, 1, f), _row3), _bspec((1, 1, f), _row3)],
        out_shape=[jax.ShapeDtypeStruct((N, 32), jnp.float32),
                   jax.ShapeDtypeStruct((NB, 1, f), jnp.float32),
                   jax.ShapeDtypeStruct((NB, 1, f), jnp.float32)],
    )(*args)


def _bn_pool_body(out_k, s, ss, g, be, pmax_o, psum_o):
    xk = _bn_relu(out_k, s, ss, g, be)
    pmax_o[...] = jnp.max(xk, axis=0, keepdims=True)[None]
    psum_o[...] = jnp.sum(xk, axis=0, keepdims=True)[None]


def _bn_pool(out_k, s, ss, g, be, f):
    return pl.pallas_call(
        _bn_pool_body,
        grid=(NB,),
        in_specs=[_bspec((BN, f), _row), _bspec((NB, 1, f), _zero3),
                  _bspec((NB, 1, f), _zero3), _bspec((1, f), _zero),
                  _bspec((1, f), _zero)],
        out_specs=[_bspec((1, 1, f), _row3), _bspec((1, 1, f), _row3)],
        out_shape=[jax.ShapeDtypeStruct((NB, 1, f), jnp.float32),
                   jax.ShapeDtypeStruct((NB, 1, f), jnp.float32)],
    )(out_k, s, ss, g, be)


def _head_body(pm1, ps1, pm2, ps2, pm3, ps3, Wc1, bc1, lng, lnb, Wc2, bc2,
               out_o):
    mx = jnp.concatenate([jnp.max(pm1[...], axis=(0, 1)),
                          jnp.max(pm2[...], axis=(0, 1)),
                          jnp.max(pm3[...], axis=(0, 1))])
    mn = jnp.concatenate([jnp.sum(ps1[...], axis=(0, 1)),
                          jnp.sum(ps2[...], axis=(0, 1)),
                          jnp.sum(ps3[...], axis=(0, 1))]) / N
    pooled = jnp.concatenate([mx, mn])[None, :]
    h = jnp.dot(pooled, Wc1[...], preferred_element_type=jnp.float32) + bc1[...]
    m = jnp.mean(h, axis=-1, keepdims=True)
    v = jnp.mean((h - m) ** 2, axis=-1, keepdims=True)
    h = jnp.maximum((h - m) * lax.rsqrt(v + 1e-5) * lng[...] + lnb[...], 0.0)
    out_o[...] = jnp.dot(h, Wc2[...], preferred_element_type=jnp.float32) \
        + bc2[...]


def _head(pm1, ps1, pm2, ps2, pm3, ps3, Wc1, bc1, lng, lnb, Wc2, bc2):
    return pl.pallas_call(
        _head_body,
        out_shape=jax.ShapeDtypeStruct((1, 6), jnp.float32),
    )(pm1, ps1, pm2, ps2, pm3, ps3, Wc1, bc1, lng, lnb, Wc2, bc2)


def kernel(adj, x, W1, b1, W2, b2, W3, b3, g1, be1, g2, be2, g3, be3,
           Wc1, bc1, lng, lnb, Wc2, bc2):
    src, dst = adj[0], adj[1]
    npad = EPAD - E
    srcp = jnp.concatenate([src, jnp.zeros((npad,), jnp.int32)])
    dstp = jnp.concatenate([dst, jnp.full((npad,), N, jnp.int32)])
    s2 = srcp * 2
    src2 = jnp.concatenate([s2, s2 + 1])

    r = lambda a: a[None, :]

    dg0, dg1 = _sc_deg(dstp)
    dis, xp16 = _prep(dg0, dg1, x)

    ag1l, ag1h = _sc_prop_e(srcp, dstp, xp16)
    W1p = jnp.pad(W1, ((0, 13), (0, 0)))
    out1, s1, ss1 = _conv_out(ag1l, ag1h, xp16, dis, W1p, r(b1), 16, 32)
    h2, pm1, ps1 = _bn_split(out1, s1, ss1, r(g1), r(be1), dis, None, 32)

    ag2l, ag2h = _sc_prop_f(src2, dstp, h2.reshape(2 * N, 16))
    out2, sm2, ss2 = _conv_cat(ag2l, ag2h, h2, dis, W2, r(b2), 64)
    h3, pm2, ps2 = _bn_split(out2, sm2, ss2, r(g2), r(be2), dis, W3, 64)

    ag3l, ag3h = _sc_prop_f(src2, dstp, h3.reshape(2 * N, 16))
    out3, s3, ss3 = _cat_add(ag3l, ag3h, h3, dis, r(b3))
    pm3, ps3 = _bn_pool(out3, s3, ss3, r(g3), r(be3), 32)

    return _head(pm1, ps1, pm2, ps2, pm3, ps3,
                 Wc1, r(bc1), r(lng), r(lnb), Wc2, r(bc2))


# speedup vs baseline: 1.1922x; 1.0139x over previous
import jax
import jax.numpy as jnp
from jax import lax
from jax.experimental import pallas as pl
from jax.experimental.pallas import tpu as pltpu
from jax.experimental.pallas import tpu_sc as plsc

N = 100000
E = 1600000
EPAD = 1638400
NACC = 100224
C = 800
D = 2
BN = 2000
NB = N // BN
ZSPAN = NACC // 16


def _sc_pass_body(mode, src_hbm, dst_hbm, tab_lo, tab_hi, out_lo, out_hi,
                  *bufs):
    sv = bufs[0:D]
    dv = bufs[D:2 * D]
    rv = bufs[2 * D:3 * D]
    acc = bufs[3 * D]
    gs = bufs[3 * D + 1:3 * D + 1 + D]
    isem = bufs[3 * D + 1 + D:3 * D + 1 + 2 * D]
    rv0 = rv[0]
    cid = lax.axis_index("c")
    sid = lax.axis_index("s")
    gather = mode != "deg"

    def _fill(val):
        def fz(i, carry):
            rv0[i] = jnp.full((16,), val, jnp.float32)
            return carry
        lax.fori_loop(0, C, fz, 0)

    _fill(0.0)
    zb = sid * ZSPAN
    for j in range(ZSPAN // C):
        pltpu.sync_copy(rv0, acc.at[pl.ds(zb + j * C, C)])
    rem = ZSPAN % C
    if rem:
        pltpu.sync_copy(rv0.at[pl.ds(0, rem)],
                        acc.at[pl.ds(zb + (ZSPAN // C) * C, rem)])
    if mode == "deg":
        _fill(1.0)
    plsc.subcore_barrier()

    if mode == "prop_f":
        ept = EPAD // 16
        ebase = sid * ept
        nch = ept // C
    else:
        w = sid * 2 + cid
        ebase = w * (EPAD // 32)
        nch = (EPAD // 32) // C
    sbase = ebase

    def start_gather(idx_ref, rows_ref, sem):
        if mode == "prop_f":
            @pl.when(cid == 0)
            def _g_lo():
                pltpu.make_async_copy(tab_lo.at[idx_ref], rows_ref, sem).start()

            @pl.when(cid == 1)
            def _g_hi():
                pltpu.make_async_copy(tab_hi.at[idx_ref], rows_ref, sem).start()
        else:
            pltpu.make_async_copy(tab_lo.at[idx_ref], rows_ref, sem).start()

    def s_slice(u):
        return src_hbm.at[pl.ds(pl.multiple_of(sbase + u * C, 8), C)]

    def d_slice(u):
        return dst_hbm.at[pl.ds(pl.multiple_of(ebase + u * C, 8), C)]

    for u in range(D - 1):
        if gather:
            pltpu.sync_copy(s_slice(u), sv[u])
        pltpu.sync_copy(d_slice(u), dv[u])
    if gather:
        for u in range(D - 1):
            start_gather(sv[u], rv[u], gs[u])
        pltpu.make_async_copy(s_slice(D - 1), sv[D - 1], isem[D - 1]).start()
    pltpu.make_async_copy(d_slice(D - 1), dv[D - 1], isem[D - 1]).start()

    def step(t, b):
        bn = (b + D - 1) % D

        @pl.when(t + D - 1 < nch)
        def _start_next_gather():
            if gather:
                pltpu.make_async_copy(s_slice(t + D - 1), sv[bn], isem[bn]).wait()
            pltpu.make_async_copy(d_slice(t + D - 1), dv[bn], isem[bn]).wait()
            if gather:
                start_gather(sv[bn], rv[bn], gs[bn])

        if gather:
            pltpu.make_async_copy(tab_lo.at[sv[b]], rv[b], gs[b]).wait()
        src_rows = rv[b] if gather else rv0
        pltpu.async_copy(src_rows, acc.at[dv[b]], isem[b], add=True).wait()

        @pl.when(t + D < nch)
        def _prefetch_idx():
            if gather:
                pltpu.make_async_copy(s_slice(t + D), sv[b], isem[b]).start()
            pltpu.make_async_copy(d_slice(t + D), dv[b], isem[b]).start()

    def rotation(p, carry):
        for b in range(D):
            step(D * p + b, b)
        return carry

    lax.fori_loop(0, nch // D, rotation, 0)
    plsc.subcore_barrier()

    zoff = pl.multiple_of(sid * ZSPAN, 8)

    @pl.when(cid == 0)
    def _wb_lo():
        pltpu.sync_copy(acc.at[pl.ds(zoff, ZSPAN)],
                        out_lo.at[pl.ds(zoff, ZSPAN)])

    @pl.when(cid == 1)
    def _wb_hi():
        pltpu.sync_copy(acc.at[pl.ds(zoff, ZSPAN)],
                        out_hi.at[pl.ds(zoff, ZSPAN)])


def _make_sc(mode):
    mesh = plsc.VectorSubcoreMesh(core_axis_name="c", subcore_axis_name="s")
    scratch = (
        [pltpu.VMEM((C,), jnp.int32)] * D +
        [pltpu.VMEM((C,), jnp.int32)] * D +
        [pltpu.VMEM((C, 16), jnp.float32)] * D +
        [pltpu.VMEM_SHARED((NACC, 16), jnp.float32)] +
        [pltpu.SemaphoreType.DMA] * (2 * D)
    )
    out_t = [jax.ShapeDtypeStruct((NACC, 16), jnp.float32)] * 2

    if mode == "deg":
        def body(dst_hbm, out_lo, out_hi, *bufs):
            _sc_pass_body(mode, None, dst_hbm, None, None, out_lo, out_hi,
                          *bufs)
    elif mode == "prop_e":
        def body(src_hbm, dst_hbm, tab_hbm, out_lo, out_hi, *bufs):
            _sc_pass_body(mode, src_hbm, dst_hbm, tab_hbm, None, out_lo,
                          out_hi, *bufs)
    else:
        def body(src_hbm, dst_hbm, tab_lo, tab_hi, out_lo, out_hi, *bufs):
            _sc_pass_body(mode, src_hbm, dst_hbm, tab_lo, tab_hi, out_lo,
                          out_hi, *bufs)

    return pl.kernel(body, mesh=mesh, out_type=out_t, scratch_types=scratch,
                     compiler_params=pltpu.CompilerParams(
                         use_tc_tiling_on_sc=False))


_sc_deg = _make_sc("deg")
_sc_prop_e = _make_sc("prop_e")
_sc_prop_f = _make_sc("prop_f")


def _bspec(shape, imap):
    return pl.BlockSpec(shape, imap)


def _row(i):
    return (i, 0)


def _zero(i):
    return (0, 0)


def _row3(i):
    return (i, 0, 0)


def _zero3(i):
    return (0, 0, 0)


def _prep_body(d0, d1, x, dis_o, xp_o):
    cnt = d0[:, 0] + d1[:, 0] + 1.0
    dis = lax.rsqrt(cnt)
    dis_o[...] = dis[:, None]
    xp_o[...] = jnp.concatenate(
        [x[...] * dis[:, None], jnp.zeros((BN, 13), jnp.float32)], axis=1)


def _prep(d0, d1, x):
    return pl.pallas_call(
        _prep_body,
        grid=(NB,),
        in_specs=[_bspec((BN, 16), _row), _bspec((BN, 16), _row),
                  _bspec((BN, 3), _row)],
        out_specs=[_bspec((BN, 1), _row), _bspec((BN, 16), _row)],
        out_shape=[jax.ShapeDtypeStruct((N, 1), jnp.float32),
                   jax.ShapeDtypeStruct((N, 16), jnp.float32)],
    )(d0, d1, x)


def _conv_out_body(a0, a1, hp, dis, W, b, out_o, s_o, ss_o):
    z = (a0[...] + a1[...] + hp[...]) * dis[...]
    o = jnp.dot(z, W[...], preferred_element_type=jnp.float32) + b[...]
    out_o[...] = o
    s_o[...] = jnp.sum(o, axis=0, keepdims=True)[None]
    ss_o[...] = jnp.sum(o * o, axis=0, keepdims=True)[None]


def _conv_out(a0, a1, hp, dis, W, b, fin, fout):
    return pl.pallas_call(
        _conv_out_body,
        grid=(NB,),
        in_specs=[_bspec((BN, fin), _row), _bspec((BN, fin), _row),
                  _bspec((BN, fin), _row), _bspec((BN, 1), _row),
                  _bspec((fin, fout), _zero), _bspec((1, fout), _zero)],
        out_specs=[_bspec((BN, fout), _row), _bspec((1, 1, fout), _row3),
                   _bspec((1, 1, fout), _row3)],
        out_shape=[jax.ShapeDtypeStruct((N, fout), jnp.float32),
                   jax.ShapeDtypeStruct((NB, 1, fout), jnp.float32),
                   jax.ShapeDtypeStruct((NB, 1, fout), jnp.float32)],
    )(a0, a1, hp, dis, W, b)


def _conv_cat_body(a0, a1, h0, h1, dis, W, b, out_o, s_o, ss_o):
    z = jnp.concatenate([a0[...] + h0[...], a1[...] + h1[...]], axis=1) * dis[...]
    o = jnp.dot(z, W[...], preferred_element_type=jnp.float32) + b[...]
    out_o[...] = o
    s_o[...] = jnp.sum(o, axis=0, keepdims=True)[None]
    ss_o[...] = jnp.sum(o * o, axis=0, keepdims=True)[None]


def _conv_cat(a0, a1, h0, h1, dis, W, b, fout):
    return pl.pallas_call(
        _conv_cat_body,
        grid=(NB,),
        in_specs=[_bspec((BN, 16), _row)] * 4 + [
            _bspec((BN, 1), _row),
            _bspec((32, fout), _zero), _bspec((1, fout), _zero)],
        out_specs=[_bspec((BN, fout), _row), _bspec((1, 1, fout), _row3),
                   _bspec((1, 1, fout), _row3)],
        out_shape=[jax.ShapeDtypeStruct((N, fout), jnp.float32),
                   jax.ShapeDtypeStruct((NB, 1, fout), jnp.float32),
                   jax.ShapeDtypeStruct((NB, 1, fout), jnp.float32)],
    )(a0, a1, h0, h1, dis, W, b)


def _cat_add_body(a0, a1, h0, h1, dis, b, out_o, s_o, ss_o):
    o = jnp.concatenate([a0[...] + h0[...], a1[...] + h1[...]], axis=1) * dis[...] \
        + b[...]
    out_o[...] = o
    s_o[...] = jnp.sum(o, axis=0, keepdims=True)[None]
    ss_o[...] = jnp.sum(o * o, axis=0, keepdims=True)[None]


def _cat_add(a0, a1, h0, h1, dis, b):
    return pl.pallas_call(
        _cat_add_body,
        grid=(NB,),
        in_specs=[_bspec((BN, 16), _row)] * 4 + [
            _bspec((BN, 1), _row), _bspec((1, 32), _zero)],
        out_specs=[_bspec((BN, 32), _row), _bspec((1, 1, 32), _row3),
                   _bspec((1, 1, 32), _row3)],
        out_shape=[jax.ShapeDtypeStruct((N, 32), jnp.float32),
                   jax.ShapeDtypeStruct((NB, 1, 32), jnp.float32),
                   jax.ShapeDtypeStruct((NB, 1, 32), jnp.float32)],
    )(a0, a1, h0, h1, dis, b)


def _bn_relu(o_ref, s, ss, g, be):
    m = jnp.sum(s[...], axis=(0, 1)) / N
    v = jnp.sum(ss[...], axis=(0, 1)) / N - m * m
    return jnp.maximum((o_ref[...] - m) * lax.rsqrt(v + 1e-5) * g[...] + be[...],
                       0.0)


def _bn_split_nw_body(out_k, s, ss, g, be, dis, h0_o, h1_o, pmax_o, psum_o):
    xk = _bn_relu(out_k, s, ss, g, be)
    pmax_o[...] = jnp.max(xk, axis=0, keepdims=True)[None]
    psum_o[...] = jnp.sum(xk, axis=0, keepdims=True)[None]
    h = xk * dis[...]
    h0_o[...] = h[:, :16]
    h1_o[...] = h[:, 16:]


def _bn_split_w_body(out_k, s, ss, g, be, dis, W, h0_o, h1_o, pmax_o, psum_o):
    xk = _bn_relu(out_k, s, ss, g, be)
    pmax_o[...] = jnp.max(xk, axis=0, keepdims=True)[None]
    psum_o[...] = jnp.sum(xk, axis=0, keepdims=True)[None]
    h = jnp.dot(xk, W[...], preferred_element_type=jnp.float32) * dis[...]
    h0_o[...] = h[:, :16]
    h1_o[...] = h[:, 16:]


def _bn_split(out_k, s, ss, g, be, dis, W, f):
    if W is None:
        body = _bn_split_nw_body
        wspecs = []
        args = (out_k, s, ss, g, be, dis)
    else:
        body = _bn_split_w_body
        wspecs = [_bspec((f, 32), _zero)]
        args = (out_k, s, ss, g, be, dis, W)
    return pl.pallas_call(
        body,
        grid=(NB,),
        in_specs=[_bspec((BN, f), _row), _bspec((NB, 1, f), _zero3),
                  _bspec((NB, 1, f), _zero3), _bspec((1, f), _zero),
                  _bspec((1, f), _zero), _bspec((BN, 1), _row)] + wspecs,
        out_specs=[_bspec((BN, 16), _row), _bspec((BN, 16), _row),
                   _bspec((1, 1, f), _row3), _bspec((1, 1, f), _row3)],
        out_shape=[jax.ShapeDtypeStruct((N, 16), jnp.float32),
                   jax.ShapeDtypeStruct((N, 16), jnp.float32),
                   jax.ShapeDtypeStruct((NB, 1, f), jnp.float32),
                   jax.ShapeDtypeStruct((NB, 1, f), jnp.float32)],
    )(*args)


def _bn_pool_body(out_k, s, ss, g, be, pmax_o, psum_o):
    xk = _bn_relu(out_k, s, ss, g, be)
    pmax_o[...] = jnp.max(xk, axis=0, keepdims=True)[None]
    psum_o[...] = jnp.sum(xk, axis=0, keepdims=True)[None]


def _bn_pool(out_k, s, ss, g, be, f):
    return pl.pallas_call(
        _bn_pool_body,
        grid=(NB,),
        in_specs=[_bspec((BN, f), _row), _bspec((NB, 1, f), _zero3),
                  _bspec((NB, 1, f), _zero3), _bspec((1, f), _zero),
                  _bspec((1, f), _zero)],
        out_specs=[_bspec((1, 1, f), _row3), _bspec((1, 1, f), _row3)],
        out_shape=[jax.ShapeDtypeStruct((NB, 1, f), jnp.float32),
                   jax.ShapeDtypeStruct((NB, 1, f), jnp.float32)],
    )(out_k, s, ss, g, be)


def _head_body(pm1, ps1, pm2, ps2, pm3, ps3, Wc1, bc1, lng, lnb, Wc2, bc2,
               out_o):
    mx = jnp.concatenate([jnp.max(pm1[...], axis=(0, 1)),
                          jnp.max(pm2[...], axis=(0, 1)),
                          jnp.max(pm3[...], axis=(0, 1))])
    mn = jnp.concatenate([jnp.sum(ps1[...], axis=(0, 1)),
                          jnp.sum(ps2[...], axis=(0, 1)),
                          jnp.sum(ps3[...], axis=(0, 1))]) / N
    pooled = jnp.concatenate([mx, mn])[None, :]
    h = jnp.dot(pooled, Wc1[...], preferred_element_type=jnp.float32) + bc1[...]
    m = jnp.mean(h, axis=-1, keepdims=True)
    v = jnp.mean((h - m) ** 2, axis=-1, keepdims=True)
    h = jnp.maximum((h - m) * lax.rsqrt(v + 1e-5) * lng[...] + lnb[...], 0.0)
    out_o[...] = jnp.dot(h, Wc2[...], preferred_element_type=jnp.float32) \
        + bc2[...]


def _head(pm1, ps1, pm2, ps2, pm3, ps3, Wc1, bc1, lng, lnb, Wc2, bc2):
    return pl.pallas_call(
        _head_body,
        out_shape=jax.ShapeDtypeStruct((1, 6), jnp.float32),
    )(pm1, ps1, pm2, ps2, pm3, ps3, Wc1, bc1, lng, lnb, Wc2, bc2)


def kernel(adj, x, W1, b1, W2, b2, W3, b3, g1, be1, g2, be2, g3, be3,
           Wc1, bc1, lng, lnb, Wc2, bc2):
    src, dst = adj[0], adj[1]
    npad = EPAD - E
    srcp = jnp.concatenate([src, jnp.zeros((npad,), jnp.int32)])
    dstp = jnp.concatenate([dst, jnp.full((npad,), N, jnp.int32)])
    r = lambda a: a[None, :]

    dg0, dg1 = _sc_deg(dstp)
    dis, xp16 = _prep(dg0, dg1, x)

    ag1l, ag1h = _sc_prop_e(srcp, dstp, xp16)
    W1p = jnp.pad(W1, ((0, 13), (0, 0)))
    out1, s1, ss1 = _conv_out(ag1l, ag1h, xp16, dis, W1p, r(b1), 16, 32)
    h2a, h2b, pm1, ps1 = _bn_split(out1, s1, ss1, r(g1), r(be1), dis, None, 32)

    ag2l, ag2h = _sc_prop_f(srcp, dstp, h2a, h2b)
    out2, sm2, ss2 = _conv_cat(ag2l, ag2h, h2a, h2b, dis, W2, r(b2), 64)
    h3a, h3b, pm2, ps2 = _bn_split(out2, sm2, ss2, r(g2), r(be2), dis, W3, 64)

    ag3l, ag3h = _sc_prop_f(srcp, dstp, h3a, h3b)
    out3, s3, ss3 = _cat_add(ag3l, ag3h, h3a, h3b, dis, r(b3))
    pm3, ps3 = _bn_pool(out3, s3, ss3, r(g3), r(be3), 32)

    return _head(pm1, ps1, pm2, ps2, pm3, ps3,
                 Wc1, r(bc1), r(lng), r(lnb), Wc2, r(bc2))
